# baseline (speedup 1.0000x reference)
import functools

import jax
import jax.numpy as jnp
from jax import lax
from jax.experimental import pallas as pl
from jax.experimental.pallas import tpu as pltpu
from jax.experimental.pallas import tpu_sc as plsc

_NB = 2048
_NC = 2
_NS = 16
_FP = 640


def _argmin_kernel(qx_ref, qy_ref, q2_ref, pts_ref, idx_out_ref,
                   run_min, run_idx):
    i = pl.program_id(0)
    nb = pts_ref.shape[1]
    px = pts_ref[0:1, :].astype(jnp.bfloat16).astype(jnp.float32)
    py = pts_ref[1:2, :].astype(jnp.bfloat16).astype(jnp.float32)
    p2 = pts_ref[2:3, :]
    qx = qx_ref[...].astype(jnp.bfloat16).astype(jnp.float32)
    qy = qy_ref[...].astype(jnp.bfloat16).astype(jnp.float32)
    q2 = q2_ref[...]
    m1 = qx * px
    m2 = qy * py
    s = m1 + m2
    t = q2 + p2
    d2 = t - 2.0 * s
    d2 = jnp.maximum(d2, 0.0)
    bmin = jnp.min(d2, axis=1, keepdims=True)
    lane = jax.lax.broadcasted_iota(jnp.int32, d2.shape, 1) + i * nb
    cand = jnp.where(d2 == bmin, lane, jnp.int32(2**30))
    bidx = jnp.min(cand, axis=1, keepdims=True)

    @pl.when(i == 0)
    def _():
        run_min[...] = bmin
        run_idx[...] = bidx

    @pl.when(i > 0)
    def _():
        upd = bmin < run_min[...]
        run_min[...] = jnp.where(upd, bmin, run_min[...])
        run_idx[...] = jnp.where(upd, bidx, run_idx[...])

    @pl.when(i == pl.num_programs(0) - 1)
    def _():
        idx_out_ref[...] = run_idx[...]


def _norm_kernel(rows_ref, out_ref):
    rows = rows_ref[...]
    s = jnp.sum(rows, axis=1, keepdims=True)
    out_ref[...] = rows[:, :out_ref.shape[1]] / (s + 1e-6)


def kernel(positions, kernels, queries):
    n = positions.shape[0]
    q = queries.shape[0]
    c, h, w = kernels.shape[1:]
    f = c * h * w

    npad = ((n + _NB - 1) // _NB) * _NB
    pad = npad - n
    big = jnp.float32(1.0e6)
    px = jnp.concatenate([positions[:, 0], jnp.full((pad,), big, jnp.float32)])
    py = jnp.concatenate([positions[:, 1], jnp.full((pad,), big, jnp.float32)])
    p2 = px * px + py * py
    pts = jnp.stack([px, py, p2])

    qx = queries[:, 0:1]
    qy = queries[:, 1:2]
    q2 = jnp.sum(queries ** 2, axis=1)[:, None]

    idx = pl.pallas_call(
        _argmin_kernel,
        grid=(npad // _NB,),
        in_specs=[
            pl.BlockSpec((q, 1), lambda i: (0, 0)),
            pl.BlockSpec((q, 1), lambda i: (0, 0)),
            pl.BlockSpec((q, 1), lambda i: (0, 0)),
            pl.BlockSpec((3, _NB), lambda i: (0, i)),
        ],
        out_specs=pl.BlockSpec((q, 1), lambda i: (0, 0)),
        out_shape=jax.ShapeDtypeStruct((q, 1), jnp.int32),
        scratch_shapes=[
            pltpu.VMEM((q, 1), jnp.float32),
            pltpu.VMEM((q, 1), jnp.int32),
        ],
    )(qx, qy, q2, pts)
    idx = idx[:, 0]

    tablep = jnp.pad(kernels.reshape(n, f), ((0, 0), (0, _FP - f)))

    nw = _NC * _NS
    b_per_w = q // nw
    e2 = b_per_w * 2
    pos1 = positions.reshape(n * 2)
    fidx2 = (idx[:, None] * 2 + jnp.arange(2, dtype=jnp.int32)[None, :]).reshape(-1)
    mesh = plsc.VectorSubcoreMesh(core_axis_name="c", subcore_axis_name="s")

    @functools.partial(
        pl.kernel,
        mesh=mesh,
        out_type=[
            jax.ShapeDtypeStruct((q, _FP), jnp.float32),
            jax.ShapeDtypeStruct((q * 2,), jnp.float32),
        ],
        scratch_types=[
            pltpu.VMEM((b_per_w,), jnp.int32),
            pltpu.VMEM((b_per_w, _FP), jnp.float32),
            pltpu.VMEM((e2,), jnp.int32),
            pltpu.VMEM((e2,), jnp.float32),
            pltpu.SemaphoreType.DMA,
            pltpu.SemaphoreType.DMA,
        ],
    )
    def _sc_gather(table_hbm, pos_hbm, idx_hbm, fidx2_hbm, rows_out, xy_out,
                   idx_v, rows_v, fidx2_v, xy_v, sem1, sem2):
        wid = lax.axis_index("s") * _NC + lax.axis_index("c")
        base = wid * b_per_w
        pltpu.sync_copy(idx_hbm.at[pl.ds(base, b_per_w)], idx_v)
        pltpu.sync_copy(fidx2_hbm.at[pl.ds(wid * e2, e2)], fidx2_v)
        cp1 = pltpu.async_copy(table_hbm.at[idx_v], rows_v, sem1)
        cp2 = pltpu.async_copy(pos_hbm.at[fidx2_v], xy_v, sem2)
        cp1.wait()
        cp2.wait()
        pltpu.sync_copy(rows_v, rows_out.at[pl.ds(base, b_per_w)])
        pltpu.sync_copy(xy_v, xy_out.at[pl.ds(wid * e2, e2)])

    rows, xy = _sc_gather(tablep, pos1, idx, fidx2)
    xy = xy.reshape(q, 2)

    psf = pl.pallas_call(
        _norm_kernel,
        grid=(1,),
        in_specs=[pl.BlockSpec((q, _FP), lambda i: (0, 0))],
        out_specs=pl.BlockSpec((q, f), lambda i: (0, 0)),
        out_shape=jax.ShapeDtypeStruct((q, f), jnp.float32),
    )(rows)

    x_sel = xy[:, 0]
    y_sel = xy[:, 1]
    psf_sel = psf.reshape(q, c, h, w)
    return (x_sel, y_sel, psf_sel)


# speedup vs baseline: 1.8943x; 1.8943x over previous
import jax
import jax.numpy as jnp
from jax.experimental import pallas as pl
from jax.experimental.pallas import tpu as pltpu

_NB = 2048
_G = 32


def _argmin_kernel(qx_ref, qy_ref, q2_ref, pts_ref, idx_out_ref,
                   run_min, run_idx):
    i = pl.program_id(0)
    nb = pts_ref.shape[1]
    px = pts_ref[0:1, :].astype(jnp.bfloat16).astype(jnp.float32)
    py = pts_ref[1:2, :].astype(jnp.bfloat16).astype(jnp.float32)
    p2 = pts_ref[2:3, :]
    qx = qx_ref[...].astype(jnp.bfloat16).astype(jnp.float32)
    qy = qy_ref[...].astype(jnp.bfloat16).astype(jnp.float32)
    q2 = q2_ref[...]
    m1 = qx * px
    m2 = qy * py
    s = m1 + m2
    t = q2 + p2
    d2 = t - 2.0 * s
    d2 = jnp.maximum(d2, 0.0)
    bmin = jnp.min(d2, axis=1, keepdims=True)
    lane = jax.lax.broadcasted_iota(jnp.int32, d2.shape, 1) + i * nb
    cand = jnp.where(d2 == bmin, lane, jnp.int32(2**30))
    bidx = jnp.min(cand, axis=1, keepdims=True)

    @pl.when(i == 0)
    def _():
        run_min[...] = bmin
        run_idx[...] = bidx

    @pl.when(i > 0)
    def _():
        upd = bmin < run_min[...]
        run_min[...] = jnp.where(upd, bmin, run_min[...])
        run_idx[...] = jnp.where(upd, bidx, run_idx[...])

    @pl.when(i == pl.num_programs(0) - 1)
    def _():
        idx_out_ref[...] = run_idx[...]


def _gather_kernel(idx_ref, *refs):
    del idx_ref
    krows = refs[:_G]
    prows = refs[_G:2 * _G]
    psf_ref = refs[2 * _G]
    xy_ref = refs[2 * _G + 1]
    for j in range(_G):
        row = krows[j][...]
        s = jnp.sum(row)
        psf_ref[j:j + 1, :, :] = row / (s + 1e-6)
        xy_ref[j:j + 1, :, :] = prows[j][...]


def kernel(positions, kernels, queries):
    n = positions.shape[0]
    q = queries.shape[0]
    c, h, w = kernels.shape[1:]
    f = c * h * w

    npad = ((n + _NB - 1) // _NB) * _NB
    pad = npad - n
    big = jnp.float32(1.0e6)
    px = jnp.concatenate([positions[:, 0], jnp.full((pad,), big, jnp.float32)])
    py = jnp.concatenate([positions[:, 1], jnp.full((pad,), big, jnp.float32)])
    p2 = px * px + py * py
    pts = jnp.stack([px, py, p2])

    qx = queries[:, 0:1]
    qy = queries[:, 1:2]
    q2 = jnp.sum(queries ** 2, axis=1)[:, None]

    idx = pl.pallas_call(
        _argmin_kernel,
        grid=(npad // _NB,),
        in_specs=[
            pl.BlockSpec((q, 1), lambda i: (0, 0)),
            pl.BlockSpec((q, 1), lambda i: (0, 0)),
            pl.BlockSpec((q, 1), lambda i: (0, 0)),
            pl.BlockSpec((3, _NB), lambda i: (0, i)),
        ],
        out_specs=pl.BlockSpec((q, 1), lambda i: (0, 0)),
        out_shape=jax.ShapeDtypeStruct((q, 1), jnp.int32),
        scratch_shapes=[
            pltpu.VMEM((q, 1), jnp.float32),
            pltpu.VMEM((q, 1), jnp.int32),
        ],
    )(qx, qy, q2, pts)
    idx = idx[:, 0]

    kflat = kernels.reshape(n, 1, f)
    pos3 = positions.reshape(n, 1, 2)
    k_specs = [
        pl.BlockSpec((1, 1, f), lambda i, idx_ref, j=j: (idx_ref[i * _G + j], 0, 0))
        for j in range(_G)
    ]
    p_specs = [
        pl.BlockSpec((1, 1, 2), lambda i, idx_ref, j=j: (idx_ref[i * _G + j], 0, 0))
        for j in range(_G)
    ]
    psf, xy = pl.pallas_call(
        _gather_kernel,
        grid_spec=pltpu.PrefetchScalarGridSpec(
            num_scalar_prefetch=1,
            grid=(q // _G,),
            in_specs=k_specs + p_specs,
            out_specs=[
                pl.BlockSpec((_G, 1, f), lambda i, idx_ref: (i, 0, 0)),
                pl.BlockSpec((_G, 1, 2), lambda i, idx_ref: (i, 0, 0)),
            ],
        ),
        out_shape=[
            jax.ShapeDtypeStruct((q, 1, f), jnp.float32),
            jax.ShapeDtypeStruct((q, 1, 2), jnp.float32),
        ],
    )(idx, *([kflat] * _G), *([pos3] * _G))

    psf_sel = psf.reshape(q, c, h, w)
    return (xy[:, 0, 0], xy[:, 0, 1], psf_sel)


# speedup vs baseline: 1.9101x; 1.0084x over previous
import jax
import jax.numpy as jnp
from jax.experimental import pallas as pl
from jax.experimental.pallas import tpu as pltpu

_NB = 2048
_G = 32


def _argmin_kernel(qx_ref, qy_ref, q2_ref, pts_ref, idx_out_ref,
                   run_min, run_idx):
    i = pl.program_id(0)
    nb = pts_ref.shape[1]
    px = pts_ref[0:1, :].astype(jnp.bfloat16).astype(jnp.float32)
    py = pts_ref[1:2, :].astype(jnp.bfloat16).astype(jnp.float32)
    p2 = pts_ref[2:3, :]
    qx = qx_ref[...].astype(jnp.bfloat16).astype(jnp.float32)
    qy = qy_ref[...].astype(jnp.bfloat16).astype(jnp.float32)
    q2 = q2_ref[...]
    m1 = qx * px
    m2 = qy * py
    s = m1 + m2
    t = q2 + p2
    d2 = t - 2.0 * s
    d2 = jnp.maximum(d2, 0.0)
    bmin = jnp.min(d2, axis=1, keepdims=True)
    lane = jax.lax.broadcasted_iota(jnp.int32, d2.shape, 1) + i * nb
    cand = jnp.where(d2 == bmin, lane, jnp.int32(2**30))
    bidx = jnp.min(cand, axis=1, keepdims=True)

    @pl.when(i == 0)
    def _():
        run_min[...] = bmin
        run_idx[...] = bidx

    @pl.when(i > 0)
    def _():
        upd = bmin < run_min[...]
        run_min[...] = jnp.where(upd, bmin, run_min[...])
        run_idx[...] = jnp.where(upd, bidx, run_idx[...])

    @pl.when(i == pl.num_programs(0) - 1)
    def _():
        idx_out_ref[...] = run_idx[...]


def _gather_kernel(idx_ref, *refs):
    del idx_ref
    krows = refs[:_G]
    prows = refs[_G:2 * _G]
    psf_ref = refs[2 * _G]
    xy_ref = refs[2 * _G + 1]
    for j in range(_G):
        psf_ref[j:j + 1, :] = krows[j][0]
        xy_ref[j:j + 1, :] = prows[j][0]


def _norm_kernel(rows_ref, out_ref):
    rows = rows_ref[...]
    s = jnp.sum(rows, axis=1, keepdims=True)
    out_ref[...] = rows / (s + 1e-6)


def kernel(positions, kernels, queries):
    n = positions.shape[0]
    q = queries.shape[0]
    c, h, w = kernels.shape[1:]
    f = c * h * w

    npad = ((n + _NB - 1) // _NB) * _NB
    pad = npad - n
    big = jnp.float32(1.0e6)
    px = jnp.concatenate([positions[:, 0], jnp.full((pad,), big, jnp.float32)])
    py = jnp.concatenate([positions[:, 1], jnp.full((pad,), big, jnp.float32)])
    p2 = px * px + py * py
    pts = jnp.stack([px, py, p2])

    qx = queries[:, 0:1]
    qy = queries[:, 1:2]
    q2 = jnp.sum(queries ** 2, axis=1)[:, None]

    idx = pl.pallas_call(
        _argmin_kernel,
        grid=(npad // _NB,),
        in_specs=[
            pl.BlockSpec((q, 1), lambda i: (0, 0)),
            pl.BlockSpec((q, 1), lambda i: (0, 0)),
            pl.BlockSpec((q, 1), lambda i: (0, 0)),
            pl.BlockSpec((3, _NB), lambda i: (0, i)),
        ],
        out_specs=pl.BlockSpec((q, 1), lambda i: (0, 0)),
        out_shape=jax.ShapeDtypeStruct((q, 1), jnp.int32),
        scratch_shapes=[
            pltpu.VMEM((q, 1), jnp.float32),
            pltpu.VMEM((q, 1), jnp.int32),
        ],
    )(qx, qy, q2, pts)
    idx = idx[:, 0]

    kflat = kernels.reshape(n, 1, f)
    pos3 = positions.reshape(n, 1, 2)
    k_specs = [
        pl.BlockSpec((1, 1, f), lambda i, idx_ref, j=j: (idx_ref[i * _G + j], 0, 0))
        for j in range(_G)
    ]
    p_specs = [
        pl.BlockSpec((1, 1, 2), lambda i, idx_ref, j=j: (idx_ref[i * _G + j], 0, 0))
        for j in range(_G)
    ]
    psf, xy = pl.pallas_call(
        _gather_kernel,
        grid_spec=pltpu.PrefetchScalarGridSpec(
            num_scalar_prefetch=1,
            grid=(q // _G,),
            in_specs=k_specs + p_specs,
            out_specs=[
                pl.BlockSpec((_G, f), lambda i, idx_ref: (i, 0)),
                pl.BlockSpec((_G, 2), lambda i, idx_ref: (i, 0)),
            ],
        ),
        out_shape=[
            jax.ShapeDtypeStruct((q, f), jnp.float32),
            jax.ShapeDtypeStruct((q, 2), jnp.float32),
        ],
    )(idx, *([kflat] * _G), *([pos3] * _G))

    psf = pl.pallas_call(
        _norm_kernel,
        grid=(1,),
        in_specs=[pl.BlockSpec((q, f), lambda i: (0, 0))],
        out_specs=pl.BlockSpec((q, f), lambda i: (0, 0)),
        out_shape=jax.ShapeDtypeStruct((q, f), jnp.float32),
    )(psf)

    psf_sel = psf.reshape(q, c, h, w)
    return (xy[:, 0], xy[:, 1], psf_sel)
